# baseline (speedup 1.0000x reference)
import functools

import jax
import jax.numpy as jnp
from jax import lax
from jax.experimental import pallas as pl
from jax.experimental.pallas import tpu as pltpu
from jax.experimental.pallas import tpu_sc as plsc

NC = 2
NS = 16
ZC = 448


def _make_sc_agg(feat, n_pad, chunk, n_win, split_edges, two_tables):
    mesh = plsc.VectorSubcoreMesh(core_axis_name="c", subcore_axis_name="s")
    zrows = n_pad // NS
    zsteps = zrows // ZC
    n_workers = NC * NS if split_edges else NS

    @functools.partial(
        pl.kernel,
        mesh=mesh,
        out_type=jax.ShapeDtypeStruct((NC * n_pad, feat), jnp.float32),
        compiler_params=pltpu.CompilerParams(use_tc_tiling_on_sc=False),
        scratch_types=[
            [pltpu.VMEM((chunk,), jnp.int32) for _ in range(2)],
            [pltpu.VMEM((chunk,), jnp.int32) for _ in range(2)],
            [pltpu.VMEM((chunk, feat), jnp.float32) for _ in range(2)],
            pltpu.VMEM((ZC, feat), jnp.float32),
            pltpu.VMEM_SHARED((n_pad, feat), jnp.float32),
            [pltpu.SemaphoreType.DMA for _ in range(2)],
            [pltpu.SemaphoreType.DMA for _ in range(2)],
            [pltpu.SemaphoreType.DMA for _ in range(2)],
        ],
    )
    def sc_agg(*args):
        if two_tables:
            (tab0_hbm, tab1_hbm, ei_hbm, z_hbm, out_hbm,
             sidx, didx, rows, stage, acc, sem_i, sem_g, sem_s) = args
        else:
            (tab0_hbm, ei_hbm, z_hbm, out_hbm,
             sidx, didx, rows, stage, acc, sem_i, sem_g, sem_s) = args
            tab1_hbm = tab0_hbm
        c = lax.axis_index("c")
        s = lax.axis_index("s")
        pltpu.sync_copy(z_hbm, stage)

        def zstep(i, carry):
            pltpu.sync_copy(stage, acc.at[pl.ds(s * zrows + i * ZC, ZC)])
            return carry

        lax.fori_loop(0, zsteps, zstep, 0)
        plsc.subcore_barrier()

        wid = c * NS + s if split_edges else s
        per = n_win // n_workers
        rem = n_win % n_workers
        base_w = wid * per + jnp.minimum(wid, rem)
        nw_t = per + jnp.where(wid < rem, 1, 0)

        e_len = n_win * chunk

        def fire_idx(p, w):
            ofs = (base_w + w) * chunk
            pltpu.async_copy(ei_hbm.at[pl.ds(ofs, chunk)], sidx[p],
                             sem_i[p])
            pltpu.async_copy(ei_hbm.at[pl.ds(e_len + ofs, chunk)], didx[p],
                             sem_i[p])

        def drain_idx(p):
            pltpu.make_async_copy(ei_hbm.at[pl.ds(0, chunk)], sidx[p],
                                  sem_i[p]).wait()
            pltpu.make_async_copy(ei_hbm.at[pl.ds(0, chunk)], didx[p],
                                  sem_i[p]).wait()

        def fire_gat(p):
            if two_tables:
                @pl.when(c == 0)
                def _():
                    pltpu.async_copy(tab0_hbm.at[sidx[p]], rows[p],
                                     sem_g[p])

                @pl.when(c == 1)
                def _():
                    pltpu.async_copy(tab1_hbm.at[sidx[p]], rows[p],
                                     sem_g[p])
            else:
                pltpu.async_copy(tab0_hbm.at[sidx[p]], rows[p], sem_g[p])

        def drain_gat(p):
            pltpu.make_async_copy(tab0_hbm.at[pl.ds(0, chunk)], rows[p],
                                  sem_g[p]).wait()

        def fire_sca(p):
            pltpu.async_copy(rows[p], acc.at[didx[p]], sem_s[p], add=True)

        def drain_sca(p):
            pltpu.make_async_copy(rows[p], acc.at[pl.ds(0, chunk)],
                                  sem_s[p]).wait()

        @pl.when(nw_t > 0)
        def _():
            fire_idx(0, 0)
            drain_idx(0)
            fire_gat(0)

        def body(i, carry):
            w1 = 2 * i + 1
            w2 = 2 * i + 2

            @pl.when(w1 < nw_t)
            def _():
                fire_idx(1, w1)

            drain_gat(0)
            fire_sca(0)

            @pl.when(w1 < nw_t)
            def _():
                drain_idx(1)
                fire_gat(1)

            @pl.when(w2 < nw_t)
            def _():
                fire_idx(0, w2)

            drain_sca(0)

            @pl.when(w1 < nw_t)
            def _():
                drain_gat(1)
                fire_sca(1)

            @pl.when(w2 < nw_t)
            def _():
                drain_idx(0)
                fire_gat(0)

            @pl.when(w1 < nw_t)
            def _():
                drain_sca(1)

            return carry

        lax.fori_loop(0, (nw_t + 1) // 2, body, 0)
        plsc.subcore_barrier()

        def wstep(i, carry):
            pltpu.sync_copy(acc.at[pl.ds(s * zrows + i * ZC, ZC)], stage)
            pltpu.sync_copy(
                stage,
                out_hbm.at[pl.ds(c * n_pad + s * zrows + i * ZC, ZC)])
            return carry

        lax.fori_loop(0, zsteps, wstep, 0)

    return sc_agg


def kernel(x, edge_index, W1, b1, W2, b2, W3, b3, W4, b4):
    n = x.shape[0]
    e = edge_index.shape[1]
    n_pad = -(-n // 2048) * 2048
    if n_pad == n:
        n_pad += 2048
    c1, c2 = 1024, 512

    ei = edge_index.astype(jnp.int32)
    e_pad = -(-e // c1) * c1
    if e_pad != e:
        ar = jnp.arange(e_pad - e, dtype=jnp.int32)
        ei = jnp.concatenate(
            [ei, jnp.stack([ar % 1024, n + ar % (n_pad - n)])], axis=1)
    eif = ei.reshape(2 * e_pad)

    xp8 = jnp.pad(jnp.concatenate(
        [x, jnp.ones((n, 1), jnp.float32)], axis=1),
        ((0, n_pad - n), (0, 4)))
    w1p = jnp.pad(W1, ((0, 5), (0, 0)))

    sc1 = _make_sc_agg(8, n_pad, c1, e_pad // c1, True, False)
    s1 = sc1(xp8, eif,
             jnp.zeros((ZC * 8 // 128, 128), jnp.float32).reshape(ZC, 8))

    xpT = xp8.T
    s1T = s1.reshape(2, n_pad, 8).transpose(0, 2, 1)

    blk = 14336
    nb = n_pad // blk
    cdim = (((0,), (0,)), ((), ()))

    def tca_body(xpT_ref, s1a_ref, s1b_ref, w1_ref, b1_ref,
                 qloT_ref, qhiT_ref, qlo_ref, qhi_ref):
        hpT = xpT_ref[...] + s1a_ref[0] + s1b_ref[0]
        qT = jnp.maximum(
            lax.dot_general(w1_ref[...], hpT, cdim,
                            preferred_element_type=jnp.float32)
            + b1_ref[...], 0.0)
        qloT_ref[...] = qT[:16]
        qhiT_ref[...] = qT[16:]
        qlo_ref[...] = qT[:16].T
        qhi_ref[...] = qT[16:].T

    qloT, qhiT, qlo, qhi = pl.pallas_call(
        tca_body,
        grid=(nb,),
        in_specs=[
            pl.BlockSpec((8, blk), lambda i: (0, i)),
            pl.BlockSpec((1, 8, blk), lambda i: (0, 0, i)),
            pl.BlockSpec((1, 8, blk), lambda i: (1, 0, i)),
            pl.BlockSpec((8, 32), lambda i: (0, 0)),
            pl.BlockSpec((32, 1), lambda i: (0, 0)),
        ],
        out_specs=[
            pl.BlockSpec((16, blk), lambda i: (0, i)),
            pl.BlockSpec((16, blk), lambda i: (0, i)),
            pl.BlockSpec((blk, 16), lambda i: (i, 0)),
            pl.BlockSpec((blk, 16), lambda i: (i, 0)),
        ],
        out_shape=[
            jax.ShapeDtypeStruct((16, n_pad), jnp.float32),
            jax.ShapeDtypeStruct((16, n_pad), jnp.float32),
            jax.ShapeDtypeStruct((n_pad, 16), jnp.float32),
            jax.ShapeDtypeStruct((n_pad, 16), jnp.float32),
        ],
    )(xpT, s1T, s1T, w1p, b1.reshape(32, 1))

    sc2 = _make_sc_agg(16, n_pad, c2, e_pad // c2, False, True)
    s2 = sc2(qlo, qhi, eif,
             jnp.zeros((ZC * 16 // 128, 128), jnp.float32).reshape(ZC, 16))
    s2r = s2.reshape(2, n_pad, 16)

    def tcb_body(qloT_ref, qhiT_ref, s2a_ref, s2b_ref, s1a_ref, s1b_ref,
                 w2a_ref, b2_ref, w3_ref, b3_ref, w4_ref, b4_ref, o_ref):
        qlT = qloT_ref[...] + s2a_ref[0].T
        qhT = qhiT_ref[...] + s2b_ref[0].T
        degT = s1a_ref[0, 3:4] + s1b_ref[0, 3:4] + 1.0
        w2a = w2a_ref[...]
        gT = (lax.dot_general(w2a[0], qlT, cdim,
                              preferred_element_type=jnp.float32)
              + lax.dot_general(w2a[1], qhT, cdim,
                                preferred_element_type=jnp.float32)
              + b2_ref[...] * degT)
        rT = jnp.maximum(
            lax.dot_general(w3_ref[...], gT, cdim,
                            preferred_element_type=jnp.float32)
            + b3_ref[...], 0.0)
        o_ref[...] = (lax.dot_general(rT, w4_ref[...], cdim,
                                      preferred_element_type=jnp.float32)
                      + b4_ref[...])

    out = pl.pallas_call(
        tcb_body,
        grid=(nb,),
        in_specs=[
            pl.BlockSpec((16, blk), lambda i: (0, i)),
            pl.BlockSpec((16, blk), lambda i: (0, i)),
            pl.BlockSpec((1, blk, 16), lambda i: (0, i, 0)),
            pl.BlockSpec((1, blk, 16), lambda i: (1, i, 0)),
            pl.BlockSpec((1, 8, blk), lambda i: (0, 0, i)),
            pl.BlockSpec((1, 8, blk), lambda i: (1, 0, i)),
            pl.BlockSpec((2, 16, 64), lambda i: (0, 0, 0)),
            pl.BlockSpec((64, 1), lambda i: (0, 0)),
            pl.BlockSpec((64, 64), lambda i: (0, 0)),
            pl.BlockSpec((64, 1), lambda i: (0, 0)),
            pl.BlockSpec((64, 128), lambda i: (0, 0)),
            pl.BlockSpec((1, 128), lambda i: (0, 0)),
        ],
        out_specs=pl.BlockSpec((blk, 128), lambda i: (i, 0)),
        out_shape=jax.ShapeDtypeStruct((n, 128), jnp.float32),
    )(qloT, qhiT, s2r, s2r, s1T, s1T,
      W2.reshape(2, 16, 64), b2.reshape(64, 1), W3, b3.reshape(64, 1),
      W4, b4.reshape(1, 128))

    return out


# speedup vs baseline: 1.0540x; 1.0540x over previous
import functools

import jax
import jax.numpy as jnp
from jax import lax
from jax.experimental import pallas as pl
from jax.experimental.pallas import tpu as pltpu
from jax.experimental.pallas import tpu_sc as plsc

NC = 2
NS = 16
ZC = 448


def _make_sc_agg(feat, n_pad, chunk, n_win, split_edges, two_tables):
    mesh = plsc.VectorSubcoreMesh(core_axis_name="c", subcore_axis_name="s")
    zrows = n_pad // NS
    zsteps = zrows // ZC
    n_workers = NC * NS if split_edges else NS

    @functools.partial(
        pl.kernel,
        mesh=mesh,
        out_type=jax.ShapeDtypeStruct((NC * n_pad, feat), jnp.float32),
        compiler_params=pltpu.CompilerParams(use_tc_tiling_on_sc=False),
        scratch_types=[
            [pltpu.VMEM((chunk,), jnp.int32) for _ in range(2)],
            [pltpu.VMEM((chunk,), jnp.int32) for _ in range(2)],
            [pltpu.VMEM((chunk, feat), jnp.float32) for _ in range(2)],
            pltpu.VMEM((ZC, feat), jnp.float32),
            pltpu.VMEM_SHARED((n_pad, feat), jnp.float32),
            [pltpu.SemaphoreType.DMA for _ in range(2)],
            [pltpu.SemaphoreType.DMA for _ in range(2)],
            [pltpu.SemaphoreType.DMA for _ in range(2)],
        ],
    )
    def sc_agg(*args):
        if two_tables:
            (tab0_hbm, tab1_hbm, ei_hbm, z_hbm, out_hbm,
             sidx, didx, rows, stage, acc, sem_i, sem_g, sem_s) = args
        else:
            (tab0_hbm, ei_hbm, z_hbm, out_hbm,
             sidx, didx, rows, stage, acc, sem_i, sem_g, sem_s) = args
            tab1_hbm = tab0_hbm
        c = lax.axis_index("c")
        s = lax.axis_index("s")
        pltpu.sync_copy(z_hbm, stage)

        def zstep(i, carry):
            pltpu.sync_copy(stage, acc.at[pl.ds(s * zrows + i * ZC, ZC)])
            return carry

        lax.fori_loop(0, zsteps, zstep, 0)
        plsc.subcore_barrier()

        wid = c * NS + s if split_edges else s
        per = n_win // n_workers
        rem = n_win % n_workers
        base_w = wid * per + jnp.minimum(wid, rem)
        nw_t = per + jnp.where(wid < rem, 1, 0)

        e_len = n_win * chunk

        def fire_idx(p, w):
            ofs = (base_w + w) * chunk
            pltpu.async_copy(ei_hbm.at[pl.ds(ofs, chunk)], sidx[p],
                             sem_i[p])
            pltpu.async_copy(ei_hbm.at[pl.ds(e_len + ofs, chunk)], didx[p],
                             sem_i[p])

        def drain_idx(p):
            pltpu.make_async_copy(ei_hbm.at[pl.ds(0, chunk)], sidx[p],
                                  sem_i[p]).wait()
            pltpu.make_async_copy(ei_hbm.at[pl.ds(0, chunk)], didx[p],
                                  sem_i[p]).wait()

        def fire_gat(p):
            if two_tables:
                @pl.when(c == 0)
                def _():
                    pltpu.async_copy(tab0_hbm.at[sidx[p]], rows[p],
                                     sem_g[p])

                @pl.when(c == 1)
                def _():
                    pltpu.async_copy(tab1_hbm.at[sidx[p]], rows[p],
                                     sem_g[p])
            else:
                pltpu.async_copy(tab0_hbm.at[sidx[p]], rows[p], sem_g[p])

        def drain_gat(p):
            pltpu.make_async_copy(tab0_hbm.at[pl.ds(0, chunk)], rows[p],
                                  sem_g[p]).wait()

        def fire_sca(p):
            pltpu.async_copy(rows[p], acc.at[didx[p]], sem_s[p], add=True)

        def drain_sca(p):
            pltpu.make_async_copy(rows[p], acc.at[pl.ds(0, chunk)],
                                  sem_s[p]).wait()

        @pl.when(nw_t > 0)
        def _():
            fire_idx(0, 0)
            drain_idx(0)
            fire_gat(0)

        def body(i, carry):
            w1 = 2 * i + 1
            w2 = 2 * i + 2

            @pl.when(w1 < nw_t)
            def _():
                fire_idx(1, w1)

            drain_gat(0)
            fire_sca(0)

            @pl.when(w1 < nw_t)
            def _():
                drain_idx(1)
                fire_gat(1)

            @pl.when(w2 < nw_t)
            def _():
                fire_idx(0, w2)

            drain_sca(0)

            @pl.when(w1 < nw_t)
            def _():
                drain_gat(1)
                fire_sca(1)

            @pl.when(w2 < nw_t)
            def _():
                drain_idx(0)
                fire_gat(0)

            @pl.when(w1 < nw_t)
            def _():
                drain_sca(1)

            return carry

        lax.fori_loop(0, (nw_t + 1) // 2, body, 0)
        plsc.subcore_barrier()

        def wstep(i, carry):
            pltpu.sync_copy(acc.at[pl.ds(s * zrows + i * ZC, ZC)], stage)
            pltpu.sync_copy(
                stage,
                out_hbm.at[pl.ds(c * n_pad + s * zrows + i * ZC, ZC)])
            return carry

        lax.fori_loop(0, zsteps, wstep, 0)

    return sc_agg


def kernel(x, edge_index, W1, b1, W2, b2, W3, b3, W4, b4):
    n = x.shape[0]
    e = edge_index.shape[1]
    n_pad = -(-n // 2048) * 2048
    if n_pad == n:
        n_pad += 2048
    c1, c2 = 1024, 640

    ei = edge_index.astype(jnp.int32)
    e_align = 5120
    e_pad = -(-e // e_align) * e_align
    if e_pad != e:
        ar = jnp.arange(e_pad - e, dtype=jnp.int32)
        ei = jnp.concatenate(
            [ei, jnp.stack([ar % 1024, n + ar % (n_pad - n)])], axis=1)
    eif = ei.reshape(2 * e_pad)

    xp8 = jnp.pad(jnp.concatenate(
        [x, jnp.ones((n, 1), jnp.float32)], axis=1),
        ((0, n_pad - n), (0, 4)))
    w1p = jnp.pad(W1, ((0, 5), (0, 0)))

    sc1 = _make_sc_agg(8, n_pad, c1, e_pad // c1, True, False)
    s1 = sc1(xp8, eif,
             jnp.zeros((ZC * 8 // 128, 128), jnp.float32).reshape(ZC, 8))

    xpT = xp8.T
    s1T = s1.reshape(2, n_pad, 8).transpose(0, 2, 1)

    blk = 14336
    nb = n_pad // blk
    cdim = (((0,), (0,)), ((), ()))

    def tca_body(xpT_ref, s1a_ref, s1b_ref, w1_ref, b1_ref,
                 qloT_ref, qhiT_ref, qlo_ref, qhi_ref):
        hpT = xpT_ref[...] + s1a_ref[0] + s1b_ref[0]
        qT = jnp.maximum(
            lax.dot_general(w1_ref[...], hpT, cdim,
                            preferred_element_type=jnp.float32)
            + b1_ref[...], 0.0)
        qloT_ref[...] = qT[:16]
        qhiT_ref[...] = qT[16:]
        qlo_ref[...] = qT[:16].T
        qhi_ref[...] = qT[16:].T

    qloT, qhiT, qlo, qhi = pl.pallas_call(
        tca_body,
        grid=(nb,),
        in_specs=[
            pl.BlockSpec((8, blk), lambda i: (0, i)),
            pl.BlockSpec((1, 8, blk), lambda i: (0, 0, i)),
            pl.BlockSpec((1, 8, blk), lambda i: (1, 0, i)),
            pl.BlockSpec((8, 32), lambda i: (0, 0)),
            pl.BlockSpec((32, 1), lambda i: (0, 0)),
        ],
        out_specs=[
            pl.BlockSpec((16, blk), lambda i: (0, i)),
            pl.BlockSpec((16, blk), lambda i: (0, i)),
            pl.BlockSpec((blk, 16), lambda i: (i, 0)),
            pl.BlockSpec((blk, 16), lambda i: (i, 0)),
        ],
        out_shape=[
            jax.ShapeDtypeStruct((16, n_pad), jnp.float32),
            jax.ShapeDtypeStruct((16, n_pad), jnp.float32),
            jax.ShapeDtypeStruct((n_pad, 16), jnp.float32),
            jax.ShapeDtypeStruct((n_pad, 16), jnp.float32),
        ],
    )(xpT, s1T, s1T, w1p, b1.reshape(32, 1))

    sc2 = _make_sc_agg(16, n_pad, c2, e_pad // c2, False, True)
    s2 = sc2(qlo, qhi, eif,
             jnp.zeros((ZC * 16 // 128, 128), jnp.float32).reshape(ZC, 16))
    s2T = s2.reshape(2, n_pad, 16).transpose(0, 2, 1)

    def tcb_body(qloT_ref, qhiT_ref, s2a_ref, s2b_ref, s1a_ref, s1b_ref,
                 w2a_ref, b2_ref, w3_ref, b3_ref, w4_ref, b4_ref, o_ref):
        qlT = qloT_ref[...] + s2a_ref[0]
        qhT = qhiT_ref[...] + s2b_ref[0]
        degT = s1a_ref[0, 3:4] + s1b_ref[0, 3:4] + 1.0
        w2a = w2a_ref[...]
        gT = (lax.dot_general(w2a[0], qlT, cdim,
                              preferred_element_type=jnp.float32)
              + lax.dot_general(w2a[1], qhT, cdim,
                                preferred_element_type=jnp.float32)
              + b2_ref[...] * degT)
        rT = jnp.maximum(
            lax.dot_general(w3_ref[...], gT, cdim,
                            preferred_element_type=jnp.float32)
            + b3_ref[...], 0.0)
        o_ref[...] = (lax.dot_general(rT, w4_ref[...], cdim,
                                      preferred_element_type=jnp.float32)
                      + b4_ref[...])

    out = pl.pallas_call(
        tcb_body,
        grid=(nb,),
        in_specs=[
            pl.BlockSpec((16, blk), lambda i: (0, i)),
            pl.BlockSpec((16, blk), lambda i: (0, i)),
            pl.BlockSpec((1, 16, blk), lambda i: (0, 0, i)),
            pl.BlockSpec((1, 16, blk), lambda i: (1, 0, i)),
            pl.BlockSpec((1, 8, blk), lambda i: (0, 0, i)),
            pl.BlockSpec((1, 8, blk), lambda i: (1, 0, i)),
            pl.BlockSpec((2, 16, 64), lambda i: (0, 0, 0)),
            pl.BlockSpec((64, 1), lambda i: (0, 0)),
            pl.BlockSpec((64, 64), lambda i: (0, 0)),
            pl.BlockSpec((64, 1), lambda i: (0, 0)),
            pl.BlockSpec((64, 128), lambda i: (0, 0)),
            pl.BlockSpec((1, 128), lambda i: (0, 0)),
        ],
        out_specs=pl.BlockSpec((blk, 128), lambda i: (i, 0)),
        out_shape=jax.ShapeDtypeStruct((n, 128), jnp.float32),
    )(qloT, qhiT, s2T, s2T, s1T, s1T,
      W2.reshape(2, 16, 64), b2.reshape(64, 1), W3, b3.reshape(64, 1),
      W4, b4.reshape(1, 128))

    return out
